# baseline (speedup 1.0000x reference)
import functools

import jax
import jax.numpy as jnp
from jax import lax
from jax.experimental import pallas as pl
from jax.experimental.pallas import tpu as pltpu
from jax.experimental.pallas import tpu_sc as plsc

VOCAB = 100000
EMB = 128
NCHAR = 128
CDIM = 16
FSIZE = 64
FWIDTH = 5
B = 64
DL = 512
QL = 32
WL = 16
NPOS = WL - FWIDTH + 1
OUT = EMB + FSIZE

NW = 32
ND = B * DL
NQ = B * QL

DW_ROWS = ND // NW // 128
DTOK = ND // NW
QTOK = NQ // NW
CCHUNK = 128


def _sc_gather(Wt, ctT, dw, qw, dcT, qcT):
    mesh = plsc.VectorSubcoreMesh(core_axis_name="c", subcore_axis_name="s")

    @functools.partial(
        pl.kernel,
        mesh=mesh,
        compiler_params=pltpu.CompilerParams(needs_layout_passes=False),
        out_type=[
            jax.ShapeDtypeStruct((ND, EMB), jnp.float32),
            jax.ShapeDtypeStruct((NQ, EMB), jnp.float32),
            jax.ShapeDtypeStruct((ND, WL * CDIM), jnp.float32),
            jax.ShapeDtypeStruct((NQ, WL * CDIM), jnp.float32),
        ],
        scratch_types=[
            pltpu.VMEM((16, 128), jnp.int32),
            pltpu.VMEM((512, EMB), jnp.float32),
            pltpu.VMEM((CDIM, NCHAR), jnp.float32),
            pltpu.VMEM((WL * CCHUNK,), jnp.int32),
            pltpu.VMEM((CCHUNK, WL * CDIM), jnp.float32),
            pltpu.SemaphoreType.DMA,
        ],
    )
    def k(w_hbm, ctT_hbm, dw_hbm, qw_hbm, dcT_flat, qcT_flat,
          wd_out, wq_out, cd_out, cq_out,
          idx_v, wrows, ctT_v, ids_v, cemb_v, sem):
        wid = lax.axis_index("s") * 2 + lax.axis_index("c")

        for s in range(2):
            pltpu.sync_copy(dw_hbm.at[pl.ds(wid * DW_ROWS + s * 4, 4)],
                            idx_v.at[pl.ds(0, 4)])
            cps = [pltpu.async_copy(w_hbm.at[idx_v.at[jj]],
                                    wrows.at[pl.ds(jj * 128, 128)], sem)
                   for jj in range(4)]
            for cp in cps:
                cp.wait()
            pltpu.sync_copy(wrows, wd_out.at[pl.ds(wid * DTOK + s * 512, 512)])

        @pl.when(wid < 16)
        def _():
            pltpu.sync_copy(qw_hbm.at[wid], idx_v.at[0])
            pltpu.async_copy(w_hbm.at[idx_v.at[0]],
                             wrows.at[pl.ds(0, 128)], sem).wait()
            pltpu.sync_copy(wrows.at[pl.ds(0, 128)],
                            wq_out.at[pl.ds(wid * 128, 128)])

        pltpu.sync_copy(ctT_hbm, ctT_v)
        lane16 = lax.iota(jnp.int32, 16)
        dvecs = [jnp.full((16,), d, jnp.int32) for d in range(CDIM)]

        def char_tokens(ntok, carry_stride):
            def tok_body(j, c):
                ids = plsc.load_gather(ids_v, [lane16 * carry_stride + j])
                for d in range(CDIM):
                    cemb_v[j, pl.ds(d * WL, WL)] = plsc.load_gather(
                        ctT_v, [dvecs[d], ids])
                return c
            lax.fori_loop(0, ntok, tok_body, 0)

        def dc_body(c, carry):
            batch = 2 * wid + c // 4
            l0 = (c % 4) * CCHUNK
            for w in range(WL):
                pltpu.sync_copy(
                    dcT_flat.at[pl.ds((batch * WL + w) * DL + l0, CCHUNK)],
                    ids_v.at[pl.ds(w * CCHUNK, CCHUNK)])
            char_tokens(CCHUNK, CCHUNK)
            pltpu.sync_copy(
                cemb_v, cd_out.at[pl.ds(batch * DL + l0, CCHUNK)])
            return carry

        lax.fori_loop(0, DTOK // CCHUNK, dc_body, 0)

        for sb in range(2):
            batch = 2 * wid + sb
            pltpu.sync_copy(qcT_flat.at[pl.ds(batch * WL * QL, WL * QL)],
                            ids_v.at[pl.ds(0, WL * QL)])
            char_tokens(QL, QL)
            pltpu.sync_copy(cemb_v.at[pl.ds(0, QL)],
                            cq_out.at[pl.ds(batch * QL, QL)])

    return k(Wt, ctT, dw, qw, dcT, qcT)


def _tc_conv(x, wemb, M, bias, nb, ltok, blk):
    def body(x_ref, w_ref, m_ref, b_ref, o_ref):
        y = jnp.dot(x_ref[...], m_ref[...],
                    preferred_element_type=jnp.float32)
        acc = y[:, 0:FSIZE]
        for p in range(1, NPOS):
            acc = jnp.maximum(acc, y[:, p * FSIZE:(p + 1) * FSIZE])
        acc = jnp.maximum(acc + b_ref[...], 0.0)
        o_ref[0, 0:EMB, :] = jnp.transpose(w_ref[...])
        o_ref[0, EMB:OUT, :] = jnp.transpose(acc)

    lpb = ltok // blk
    return pl.pallas_call(
        body,
        grid=(nb * lpb,),
        in_specs=[
            pl.BlockSpec((blk, WL * CDIM), lambda i: (i, 0)),
            pl.BlockSpec((blk, EMB), lambda i: (i, 0)),
            pl.BlockSpec((WL * CDIM, NPOS * FSIZE), lambda i: (0, 0)),
            pl.BlockSpec((1, FSIZE), lambda i: (0, 0)),
        ],
        out_specs=pl.BlockSpec(
            (1, OUT, blk), lambda i: (i // lpb, 0, i % lpb)),
        out_shape=jax.ShapeDtypeStruct((nb, OUT, ltok), jnp.float32),
    )(x, wemb, M, bias)


def _tc_conv_tokmajor(x, wemb, M, bias, nb, ltok, blk):
    bb = blk // ltok

    def body(x_ref, w_ref, m_ref, b_ref, o_ref):
        y = jnp.dot(x_ref[...], m_ref[...],
                    preferred_element_type=jnp.float32)
        acc = y[:, 0:FSIZE]
        for p in range(1, NPOS):
            acc = jnp.maximum(acc, y[:, p * FSIZE:(p + 1) * FSIZE])
        acc = jnp.maximum(acc + b_ref[...], 0.0)
        o_ref[...] = jnp.concatenate(
            [w_ref[...], acc], axis=1).reshape(bb, ltok, OUT)

    return pl.pallas_call(
        body,
        grid=(nb // bb,),
        in_specs=[
            pl.BlockSpec((blk, WL * CDIM), lambda i: (i, 0)),
            pl.BlockSpec((blk, EMB), lambda i: (i, 0)),
            pl.BlockSpec((WL * CDIM, NPOS * FSIZE), lambda i: (0, 0)),
            pl.BlockSpec((1, FSIZE), lambda i: (0, 0)),
        ],
        out_specs=pl.BlockSpec((bb, ltok, OUT), lambda i: (i, 0, 0)),
        out_shape=jax.ShapeDtypeStruct((nb, ltok, OUT), jnp.float32),
    )(x, wemb, M, bias)


def _build_band(conv_w):
    wct = jnp.transpose(conv_w[:, :, 0, :], (2, 1, 0))
    eyes = jnp.stack([jnp.eye(WL, NPOS, k=-d, dtype=jnp.float32)
                      for d in range(FWIDTH)])
    m4 = jnp.einsum("dwp,dcf->cwpf", eyes, wct)
    return m4.reshape(WL * CDIM, NPOS * FSIZE)


def kernel(doc_w, doc_c, qry_w, qry_c, k_layer, K, W, char_table, conv_w, conv_b):
    dw = doc_w.astype(jnp.int32).reshape(ND // 128, 128)
    qw = qry_w.astype(jnp.int32).reshape(NQ // 128, 128)
    dcT = jnp.transpose(doc_c.astype(jnp.int32), (0, 2, 1)).reshape(B * WL * DL)
    qcT = jnp.transpose(qry_c.astype(jnp.int32), (0, 2, 1)).reshape(B * WL * QL)
    Wt = W.astype(jnp.float32)
    ctT = char_table.astype(jnp.float32).T

    wd, wq, cd, cq = _sc_gather(Wt, ctT, dw, qw, dcT, qcT)

    M = _build_band(conv_w.astype(jnp.float32))
    bias = conv_b.astype(jnp.float32).reshape(1, FSIZE)

    outd = _tc_conv(cd, wd, M, bias, B, DL, 512)
    outq = _tc_conv_tokmajor(cq, wq, M, bias, B, QL, 512)
    return jnp.transpose(outd, (0, 2, 1)), outq


# speedup vs baseline: 1.2622x; 1.2622x over previous
import functools

import jax
import jax.numpy as jnp
from jax import lax
from jax.experimental import pallas as pl
from jax.experimental.pallas import tpu as pltpu
from jax.experimental.pallas import tpu_sc as plsc

VOCAB = 100000
EMB = 128
NCHAR = 128
CDIM = 16
FSIZE = 64
FWIDTH = 5
B = 64
DL = 512
QL = 32
WL = 16
NPOS = WL - FWIDTH + 1
OUT = EMB + FSIZE

NW = 32
ND = B * DL
NQ = B * QL

DW_ROWS = ND // NW // 128
DTOK = ND // NW
QTOK = NQ // NW
CCHUNK = 128


def _sc_gather(Wt, ctT, dw, qw, dcT, qcT):
    mesh = plsc.VectorSubcoreMesh(core_axis_name="c", subcore_axis_name="s")

    @functools.partial(
        pl.kernel,
        mesh=mesh,
        compiler_params=pltpu.CompilerParams(needs_layout_passes=False),
        out_type=[
            jax.ShapeDtypeStruct((ND, EMB), jnp.float32),
            jax.ShapeDtypeStruct((NQ, EMB), jnp.float32),
            jax.ShapeDtypeStruct((ND, WL * CDIM), jnp.float32),
            jax.ShapeDtypeStruct((NQ, WL * CDIM), jnp.float32),
        ],
        scratch_types=[
            pltpu.VMEM((16, 128), jnp.int32),
            pltpu.VMEM((512, EMB), jnp.float32),
            pltpu.VMEM((CDIM, NCHAR), jnp.float32),
            pltpu.VMEM((WL, CCHUNK), jnp.int32),
            pltpu.VMEM((WL * QL,), jnp.int32),
            pltpu.VMEM((CCHUNK, WL * CDIM), jnp.float32),
            pltpu.SemaphoreType.DMA,
        ],
    )
    def k(w_hbm, ctT_hbm, dw_hbm, qw_hbm, dcT_flat, qcT_flat,
          wd_out, wq_out, cd_out, cq_out,
          idx_v, wrows, ctT_v, ids_v, qids_v, cemb_v, sem):
        wid = lax.axis_index("s") * 2 + lax.axis_index("c")

        for s in range(2):
            pltpu.sync_copy(dw_hbm.at[pl.ds(wid * DW_ROWS + s * 4, 4)],
                            idx_v.at[pl.ds(0, 4)])
            cps = [pltpu.async_copy(w_hbm.at[idx_v.at[jj]],
                                    wrows.at[pl.ds(jj * 128, 128)], sem)
                   for jj in range(4)]
            for cp in cps:
                cp.wait()
            pltpu.sync_copy(wrows, wd_out.at[pl.ds(wid * DTOK + s * 512, 512)])

        @pl.when(wid < 16)
        def _():
            pltpu.sync_copy(qw_hbm.at[wid], idx_v.at[0])
            pltpu.async_copy(w_hbm.at[idx_v.at[0]],
                             wrows.at[pl.ds(0, 128)], sem).wait()
            pltpu.sync_copy(wrows.at[pl.ds(0, 128)],
                            wq_out.at[pl.ds(wid * 128, 128)])

        pltpu.sync_copy(ctT_hbm, ctT_v)
        lane16 = lax.iota(jnp.int32, 16)
        dvecs = [jnp.full((16,), d, jnp.int32) for d in range(CDIM)]

        def char_tokens(ntok):
            def tok_body(j, c):
                ids = plsc.load_gather(ids_v, [lane16, jnp.full((16,), j,
                                                                jnp.int32)])
                for d in range(CDIM):
                    cemb_v[j, pl.ds(d * WL, WL)] = plsc.load_gather(
                        ctT_v, [dvecs[d], ids])
                return c
            lax.fori_loop(0, ntok, tok_body, 0)

        def dc_body(c, carry):
            batch = 2 * wid + c // 4
            l0 = (c % 4) * CCHUNK
            pltpu.sync_copy(
                dcT_flat.at[pl.ds(batch * WL, WL), pl.ds(l0, CCHUNK)],
                ids_v)
            char_tokens(CCHUNK)
            pltpu.sync_copy(
                cemb_v, cd_out.at[pl.ds(batch * DL + l0, CCHUNK)])
            return carry

        lax.fori_loop(0, DTOK // CCHUNK, dc_body, 0)

        def qry_tokens():
            def tok_body(j, c):
                ids = plsc.load_gather(qids_v, [lane16 * QL + j])
                for d in range(CDIM):
                    cemb_v[j, pl.ds(d * WL, WL)] = plsc.load_gather(
                        ctT_v, [dvecs[d], ids])
                return c
            lax.fori_loop(0, QL, tok_body, 0)

        for sb in range(2):
            batch = 2 * wid + sb
            pltpu.sync_copy(qcT_flat.at[pl.ds(batch * WL * QL, WL * QL)],
                            qids_v)
            qry_tokens()
            pltpu.sync_copy(cemb_v.at[pl.ds(0, QL)],
                            cq_out.at[pl.ds(batch * QL, QL)])

    return k(Wt, ctT, dw, qw, dcT, qcT)


def _tc_conv(x, wemb, M, bias, nb, ltok, blk):
    def body(x_ref, w_ref, m_ref, b_ref, o_ref):
        y = jnp.dot(x_ref[...], m_ref[...],
                    preferred_element_type=jnp.float32)
        acc = y[:, 0:FSIZE]
        for p in range(1, NPOS):
            acc = jnp.maximum(acc, y[:, p * FSIZE:(p + 1) * FSIZE])
        acc = jnp.maximum(acc + b_ref[...], 0.0)
        o_ref[0, 0:EMB, :] = jnp.transpose(w_ref[...])
        o_ref[0, EMB:OUT, :] = jnp.transpose(acc)

    lpb = ltok // blk
    return pl.pallas_call(
        body,
        grid=(nb * lpb,),
        in_specs=[
            pl.BlockSpec((blk, WL * CDIM), lambda i: (i, 0)),
            pl.BlockSpec((blk, EMB), lambda i: (i, 0)),
            pl.BlockSpec((WL * CDIM, NPOS * FSIZE), lambda i: (0, 0)),
            pl.BlockSpec((1, FSIZE), lambda i: (0, 0)),
        ],
        out_specs=pl.BlockSpec(
            (1, OUT, blk), lambda i: (i // lpb, 0, i % lpb)),
        out_shape=jax.ShapeDtypeStruct((nb, OUT, ltok), jnp.float32),
    )(x, wemb, M, bias)


def _tc_conv_tokmajor(x, wemb, M, bias, nb, ltok, blk):
    bb = blk // ltok

    def body(x_ref, w_ref, m_ref, b_ref, o_ref):
        y = jnp.dot(x_ref[...], m_ref[...],
                    preferred_element_type=jnp.float32)
        acc = y[:, 0:FSIZE]
        for p in range(1, NPOS):
            acc = jnp.maximum(acc, y[:, p * FSIZE:(p + 1) * FSIZE])
        acc = jnp.maximum(acc + b_ref[...], 0.0)
        o_ref[...] = jnp.concatenate(
            [w_ref[...], acc], axis=1).reshape(bb, ltok, OUT)

    return pl.pallas_call(
        body,
        grid=(nb // bb,),
        in_specs=[
            pl.BlockSpec((blk, WL * CDIM), lambda i: (i, 0)),
            pl.BlockSpec((blk, EMB), lambda i: (i, 0)),
            pl.BlockSpec((WL * CDIM, NPOS * FSIZE), lambda i: (0, 0)),
            pl.BlockSpec((1, FSIZE), lambda i: (0, 0)),
        ],
        out_specs=pl.BlockSpec((bb, ltok, OUT), lambda i: (i, 0, 0)),
        out_shape=jax.ShapeDtypeStruct((nb, ltok, OUT), jnp.float32),
    )(x, wemb, M, bias)


def _build_band(conv_w):
    wct = jnp.transpose(conv_w[:, :, 0, :], (2, 1, 0))
    eyes = jnp.stack([jnp.eye(WL, NPOS, k=-d, dtype=jnp.float32)
                      for d in range(FWIDTH)])
    m4 = jnp.einsum("dwp,dcf->cwpf", eyes, wct)
    return m4.reshape(WL * CDIM, NPOS * FSIZE)


def kernel(doc_w, doc_c, qry_w, qry_c, k_layer, K, W, char_table, conv_w, conv_b):
    dw = doc_w.astype(jnp.int32).reshape(ND // 128, 128)
    qw = qry_w.astype(jnp.int32).reshape(NQ // 128, 128)
    dcT = jnp.transpose(doc_c.astype(jnp.int32), (0, 2, 1)).reshape(B * WL, DL)
    qcT = jnp.transpose(qry_c.astype(jnp.int32), (0, 2, 1)).reshape(B * WL * QL)
    Wt = W.astype(jnp.float32)
    ctT = char_table.astype(jnp.float32).T

    wd, wq, cd, cq = _sc_gather(Wt, ctT, dw, qw, dcT, qcT)

    M = _build_band(conv_w.astype(jnp.float32))
    bias = conv_b.astype(jnp.float32).reshape(1, FSIZE)

    outd = _tc_conv(cd, wd, M, bias, B, DL, 512)
    outq = _tc_conv_tokmajor(cq, wq, M, bias, B, QL, 512)
    return jnp.transpose(outd, (0, 2, 1)), outq


# speedup vs baseline: 1.5179x; 1.2025x over previous
import functools

import jax
import jax.numpy as jnp
from jax import lax
from jax.experimental import pallas as pl
from jax.experimental.pallas import tpu as pltpu
from jax.experimental.pallas import tpu_sc as plsc

VOCAB = 100000
EMB = 128
NCHAR = 128
CDIM = 16
FSIZE = 64
FWIDTH = 5
B = 64
DL = 512
QL = 32
WL = 16
NPOS = WL - FWIDTH + 1
OUT = EMB + FSIZE

NW = 32
ND = B * DL
NQ = B * QL

DW_ROWS = ND // NW // 128
DTOK = ND // NW
QTOK = NQ // NW
CCHUNK = 128


def _sc_gather(Wt, ctT, dw, qw, dcT, qcT):
    mesh = plsc.VectorSubcoreMesh(core_axis_name="c", subcore_axis_name="s")

    @functools.partial(
        pl.kernel,
        mesh=mesh,
        compiler_params=pltpu.CompilerParams(needs_layout_passes=False),
        out_type=[
            jax.ShapeDtypeStruct((ND, EMB), jnp.float32),
            jax.ShapeDtypeStruct((NQ, EMB), jnp.float32),
            jax.ShapeDtypeStruct((ND, WL * CDIM), jnp.float32),
            jax.ShapeDtypeStruct((NQ, WL * CDIM), jnp.float32),
        ],
        scratch_types=[
            pltpu.VMEM((8, 128), jnp.int32),
            pltpu.VMEM((2, 128, EMB), jnp.float32),
            pltpu.VMEM((CDIM, NCHAR), jnp.float32),
            pltpu.VMEM((2, WL, CCHUNK), jnp.int32),
            pltpu.VMEM((WL * QL,), jnp.int32),
            pltpu.VMEM((64,), jnp.int32),
            pltpu.VMEM((2, CCHUNK, WL * CDIM), jnp.float32),
            pltpu.SemaphoreType.DMA,
            pltpu.SemaphoreType.DMA,
            pltpu.SemaphoreType.DMA,
            pltpu.SemaphoreType.DMA,
        ],
    )
    def k(w_hbm, ctT_hbm, dw_hbm, qw_hbm, dcT_flat, qcT_flat,
          wd_out, wq_out, cd_out, cq_out,
          idx_v, wrows, ctT_v, ids_v, qids_v, qwi_v, cemb_v,
          sem_g, sem_i, sem_wo, sem_co):
        wid = lax.axis_index("s") * 2 + lax.axis_index("c")

        pltpu.sync_copy(ctT_hbm, ctT_v)
        pltpu.sync_copy(dw_hbm.at[pl.ds(wid * DW_ROWS, 8)], idx_v)
        lane16 = lax.iota(jnp.int32, 16)
        dvecs = [jnp.full((16,), d, jnp.int32) for d in range(CDIM)]

        def ids_copy(c, buf):
            batch = 2 * wid + c // 4
            l0 = (c % 4) * CCHUNK
            return pltpu.async_copy(
                dcT_flat.at[pl.ds(batch * WL, WL), pl.ds(l0, CCHUNK)],
                ids_v.at[buf], sem_i)

        def char_tokens(buf):
            def tok_body(j, c):
                ids = plsc.load_gather(
                    ids_v.at[buf], [lane16, jnp.full((16,), j, jnp.int32)])
                for d in range(CDIM):
                    cemb_v[buf, j, pl.ds(d * WL, WL)] = plsc.load_gather(
                        ctT_v, [dvecs[d], ids])
                return c
            lax.fori_loop(0, CCHUNK, tok_body, 0)

        ids_copy(0, 0).wait()
        ids_pf = ids_copy(1, 1)
        gat = {}
        wrt_w = {}
        wrt_c = {}
        for c in range(8):
            h = c % 2
            if c >= 2:
                wrt_w[c - 2].wait()
                wrt_c[c - 2].wait()
            gat[c] = pltpu.async_copy(w_hbm.at[idx_v.at[c]],
                                      wrows.at[h], sem_g)
            if c >= 1:
                ids_pf.wait()
                if c < 7:
                    ids_pf = ids_copy(c + 1, (c + 1) % 2)
            char_tokens(h)
            batch = 2 * wid + c // 4
            l0 = (c % 4) * CCHUNK
            wrt_c[c] = pltpu.async_copy(
                cemb_v.at[h], cd_out.at[pl.ds(batch * DL + l0, CCHUNK)],
                sem_co)
            gat[c].wait()
            wrt_w[c] = pltpu.async_copy(
                wrows.at[h], wd_out.at[pl.ds(wid * DTOK + c * 128, 128)],
                sem_wo)
        wrt_w[6].wait()
        wrt_c[6].wait()
        wrt_w[7].wait()
        wrt_c[7].wait()

        pltpu.sync_copy(qw_hbm.at[pl.ds(wid * 64, 64)], qwi_v)
        qw_gat = pltpu.async_copy(w_hbm.at[qwi_v],
                                  wrows.at[0, pl.ds(0, 64)], sem_g)

        def qry_tokens(buf, base):
            def tok_body(j, c):
                ids = plsc.load_gather(qids_v, [lane16 * QL + j])
                for d in range(CDIM):
                    cemb_v[buf, base + j, pl.ds(d * WL, WL)] = (
                        plsc.load_gather(ctT_v, [dvecs[d], ids]))
                return c
            lax.fori_loop(0, QL, tok_body, 0)

        for sb in range(2):
            batch = 2 * wid + sb
            pltpu.sync_copy(qcT_flat.at[pl.ds(batch * WL * QL, WL * QL)],
                            qids_v)
            qry_tokens(0, sb * QL)
        pltpu.sync_copy(cemb_v.at[0, pl.ds(0, 2 * QL)],
                        cq_out.at[pl.ds(2 * wid * QL, 2 * QL)])

        qw_gat.wait()
        pltpu.sync_copy(wrows.at[0, pl.ds(0, 64)],
                        wq_out.at[pl.ds(wid * 64, 64)])

    return k(Wt, ctT, dw, qw, dcT, qcT)


def _tc_conv(x, wemb, M, bias, nb, ltok, blk):
    def body(x_ref, w_ref, m_ref, b_ref, o_ref):
        y = lax.dot_general(
            m_ref[...], x_ref[...].astype(jnp.bfloat16),
            (((1,), (1,)), ((), ())),
            preferred_element_type=jnp.float32)
        acc = y[0:FSIZE, :]
        for p in range(1, NPOS):
            acc = jnp.maximum(acc, y[p * FSIZE:(p + 1) * FSIZE, :])
        acc = jnp.maximum(acc + b_ref[...], 0.0)
        o_ref[0, 0:EMB, :] = jnp.transpose(w_ref[...])
        o_ref[0, EMB:OUT, :] = acc

    lpb = ltok // blk
    return pl.pallas_call(
        body,
        grid=(nb * lpb,),
        in_specs=[
            pl.BlockSpec((blk, WL * CDIM), lambda i: (i, 0)),
            pl.BlockSpec((blk, EMB), lambda i: (i, 0)),
            pl.BlockSpec((NPOS * FSIZE, WL * CDIM), lambda i: (0, 0)),
            pl.BlockSpec((FSIZE, 1), lambda i: (0, 0)),
        ],
        out_specs=pl.BlockSpec(
            (1, OUT, blk), lambda i: (i // lpb, 0, i % lpb)),
        out_shape=jax.ShapeDtypeStruct((nb, OUT, ltok), jnp.float32),
    )(x, wemb, M.T, bias.reshape(FSIZE, 1))


def _tc_conv_tokmajor(x, wemb, M, bias, nb, ltok, blk):
    bb = blk // ltok

    def body(x_ref, w_ref, m_ref, b_ref, o_ref):
        y = jnp.dot(x_ref[...].astype(jnp.bfloat16), m_ref[...],
                    preferred_element_type=jnp.float32)
        acc = y[:, 0:FSIZE]
        for p in range(1, NPOS):
            acc = jnp.maximum(acc, y[:, p * FSIZE:(p + 1) * FSIZE])
        acc = jnp.maximum(acc + b_ref[...], 0.0)
        o_ref[...] = jnp.concatenate(
            [w_ref[...], acc], axis=1).reshape(bb, ltok, OUT)

    return pl.pallas_call(
        body,
        grid=(nb // bb,),
        in_specs=[
            pl.BlockSpec((blk, WL * CDIM), lambda i: (i, 0)),
            pl.BlockSpec((blk, EMB), lambda i: (i, 0)),
            pl.BlockSpec((WL * CDIM, NPOS * FSIZE), lambda i: (0, 0)),
            pl.BlockSpec((1, FSIZE), lambda i: (0, 0)),
        ],
        out_specs=pl.BlockSpec((bb, ltok, OUT), lambda i: (i, 0, 0)),
        out_shape=jax.ShapeDtypeStruct((nb, ltok, OUT), jnp.float32),
    )(x, wemb, M, bias)


def _build_band(conv_w):
    wct = jnp.transpose(conv_w[:, :, 0, :], (2, 1, 0))
    eyes = jnp.stack([jnp.eye(WL, NPOS, k=-d, dtype=jnp.float32)
                      for d in range(FWIDTH)])
    m4 = jnp.einsum("dwp,dcf->cwpf", eyes, wct)
    return m4.reshape(WL * CDIM, NPOS * FSIZE)


def kernel(doc_w, doc_c, qry_w, qry_c, k_layer, K, W, char_table, conv_w, conv_b):
    dw = doc_w.astype(jnp.int32).reshape(ND // 128, 128)
    qw = qry_w.astype(jnp.int32).reshape(NQ)
    dcT = jnp.transpose(doc_c.astype(jnp.int32), (0, 2, 1)).reshape(B * WL, DL)
    qcT = jnp.transpose(qry_c.astype(jnp.int32), (0, 2, 1)).reshape(B * WL * QL)
    Wt = W.astype(jnp.float32)
    ctT = char_table.astype(jnp.float32).T

    wd, wq, cd, cq = _sc_gather(Wt, ctT, dw, qw, dcT, qcT)

    M = _build_band(conv_w.astype(jnp.float32)).astype(jnp.bfloat16)
    bias = conv_b.astype(jnp.float32).reshape(1, FSIZE)

    outd = _tc_conv(cd, wd, M, bias, B, DL, 512)
    outq = _tc_conv_tokmajor(cq, wq, M, bias, B, QL, 512)
    return jnp.transpose(outd, (0, 2, 1)), outq


# speedup vs baseline: 1.7242x; 1.1360x over previous
import functools

import jax
import jax.numpy as jnp
from jax import lax
from jax.experimental import pallas as pl
from jax.experimental.pallas import tpu as pltpu
from jax.experimental.pallas import tpu_sc as plsc

VOCAB = 100000
EMB = 128
NCHAR = 128
CDIM = 16
FSIZE = 64
FWIDTH = 5
B = 64
DL = 512
QL = 32
WL = 16
NPOS = WL - FWIDTH + 1
OUT = EMB + FSIZE

NW = 32
ND = B * DL
NQ = B * QL
BH = B // 2
NDH = BH * DL
CCHUNK = 128
NCH = DL // CCHUNK


def _sc_gather_half(Wt, ctT, dw, dcT, qw=None, qcT=None):
    with_qry = qw is not None
    mesh = plsc.VectorSubcoreMesh(core_axis_name="c", subcore_axis_name="s")
    out_type = [
        jax.ShapeDtypeStruct((NDH, EMB), jnp.float32),
        jax.ShapeDtypeStruct((NDH, WL * CDIM), jnp.float32),
    ]
    if with_qry:
        out_type += [
            jax.ShapeDtypeStruct((NQ, EMB), jnp.float32),
            jax.ShapeDtypeStruct((NQ, WL * CDIM), jnp.float32),
        ]

    def k(*refs):
        if with_qry:
            (w_hbm, ctT_hbm, dw_hbm, dcT_flat, qw_hbm, qcT_flat,
             wd_out, cd_out, wq_out, cq_out,
             idx_v, wrows, ctT_v, ids_v, qids_v, qwi_v, cemb_v,
             sem_g, sem_i, sem_wo, sem_co) = refs
        else:
            (w_hbm, ctT_hbm, dw_hbm, dcT_flat,
             wd_out, cd_out,
             idx_v, wrows, ctT_v, ids_v, qids_v, qwi_v, cemb_v,
             sem_g, sem_i, sem_wo, sem_co) = refs
        wid = lax.axis_index("s") * 2 + lax.axis_index("c")

        pltpu.sync_copy(ctT_hbm, ctT_v)
        pltpu.sync_copy(dw_hbm.at[pl.ds(wid * NCH, NCH)], idx_v)
        lane16 = lax.iota(jnp.int32, 16)
        dvecs = [jnp.full((16,), d, jnp.int32) for d in range(CDIM)]

        def ids_copy(c, buf):
            return pltpu.async_copy(
                dcT_flat.at[pl.ds(wid * WL, WL), pl.ds(c * CCHUNK, CCHUNK)],
                ids_v.at[buf], sem_i)

        def char_tokens(buf):
            def tok_body(j, c):
                ids = plsc.load_gather(
                    ids_v.at[buf], [lane16, jnp.full((16,), j, jnp.int32)])
                for d in range(CDIM):
                    cemb_v[buf, j, pl.ds(d * WL, WL)] = plsc.load_gather(
                        ctT_v, [dvecs[d], ids])
                return c
            lax.fori_loop(0, CCHUNK, tok_body, 0)

        ids_copy(0, 0).wait()
        ids_pf = ids_copy(1, 1)
        gat = {}
        wrt_w = {}
        wrt_c = {}
        for c in range(NCH):
            h = c % 2
            if c >= 2:
                wrt_w[c - 2].wait()
                wrt_c[c - 2].wait()
            gat[c] = pltpu.async_copy(w_hbm.at[idx_v.at[c]],
                                      wrows.at[h], sem_g)
            if c >= 1:
                ids_pf.wait()
                if c < NCH - 1:
                    ids_pf = ids_copy(c + 1, (c + 1) % 2)
            char_tokens(h)
            off = wid * DL + c * CCHUNK
            wrt_c[c] = pltpu.async_copy(
                cemb_v.at[h], cd_out.at[pl.ds(off, CCHUNK)], sem_co)
            gat[c].wait()
            wrt_w[c] = pltpu.async_copy(
                wrows.at[h], wd_out.at[pl.ds(off, CCHUNK)], sem_wo)
        for c in range(NCH - 2, NCH):
            wrt_w[c].wait()
            wrt_c[c].wait()

        if with_qry:
            pltpu.sync_copy(qw_hbm.at[pl.ds(wid * 64, 64)], qwi_v)
            qw_gat = pltpu.async_copy(w_hbm.at[qwi_v],
                                      wrows.at[0, pl.ds(0, 64)], sem_g)

            def qry_tokens(base):
                def tok_body(j, c):
                    ids = plsc.load_gather(qids_v, [lane16 * QL + j])
                    for d in range(CDIM):
                        cemb_v[0, base + j, pl.ds(d * WL, WL)] = (
                            plsc.load_gather(ctT_v, [dvecs[d], ids]))
                    return c
                lax.fori_loop(0, QL, tok_body, 0)

            for sb in range(2):
                batch = 2 * wid + sb
                pltpu.sync_copy(
                    qcT_flat.at[pl.ds(batch * WL * QL, WL * QL)], qids_v)
                qry_tokens(sb * QL)
            pltpu.sync_copy(cemb_v.at[0, pl.ds(0, 2 * QL)],
                            cq_out.at[pl.ds(2 * wid * QL, 2 * QL)])

            qw_gat.wait()
            pltpu.sync_copy(wrows.at[0, pl.ds(0, 64)],
                            wq_out.at[pl.ds(wid * 64, 64)])

    kk = pl.kernel(
        k,
        mesh=mesh,
        compiler_params=pltpu.CompilerParams(needs_layout_passes=False),
        out_type=out_type,
        scratch_types=[
            pltpu.VMEM((NCH, 128), jnp.int32),
            pltpu.VMEM((2, 128, EMB), jnp.float32),
            pltpu.VMEM((CDIM, NCHAR), jnp.float32),
            pltpu.VMEM((2, WL, CCHUNK), jnp.int32),
            pltpu.VMEM((WL * QL,), jnp.int32),
            pltpu.VMEM((64,), jnp.int32),
            pltpu.VMEM((2, CCHUNK, WL * CDIM), jnp.float32),
            pltpu.SemaphoreType.DMA,
            pltpu.SemaphoreType.DMA,
            pltpu.SemaphoreType.DMA,
            pltpu.SemaphoreType.DMA,
        ],
    )
    if with_qry:
        return kk(Wt, ctT, dw, dcT, qw, qcT)
    return kk(Wt, ctT, dw, dcT)


def _tc_conv(x, wemb, Mt, bias, nb, b_off, nb_total, prev=None):
    def body(x_ref, w_ref, m_ref, b_ref, *rest):
        o_ref = rest[-1]
        y = lax.dot_general(
            m_ref[...], x_ref[...].astype(jnp.bfloat16),
            (((1,), (1,)), ((), ())),
            preferred_element_type=jnp.float32)
        acc = y[0:FSIZE, :]
        for p in range(1, NPOS):
            acc = jnp.maximum(acc, y[p * FSIZE:(p + 1) * FSIZE, :])
        acc = jnp.maximum(acc + b_ref[...], 0.0)
        o_ref[0, 0:EMB, :] = jnp.transpose(w_ref[...])
        o_ref[0, EMB:OUT, :] = acc

    in_specs = [
        pl.BlockSpec((DL, WL * CDIM), lambda i: (i, 0)),
        pl.BlockSpec((DL, EMB), lambda i: (i, 0)),
        pl.BlockSpec((NPOS * FSIZE, WL * CDIM), lambda i: (0, 0)),
        pl.BlockSpec((FSIZE, 1), lambda i: (0, 0)),
    ]
    args = [x, wemb, Mt, bias]
    aliases = {}
    if prev is not None:
        in_specs.append(pl.BlockSpec(memory_space=pl.ANY))
        args.append(prev)
        aliases = {4: 0}
    return pl.pallas_call(
        body,
        grid=(nb,),
        in_specs=in_specs,
        out_specs=pl.BlockSpec((1, OUT, DL), lambda i: (i + b_off, 0, 0)),
        out_shape=jax.ShapeDtypeStruct((nb_total, OUT, DL), jnp.float32),
        input_output_aliases=aliases,
    )(*args)


def _tc_conv_qry(x, wemb, Mt, bias):
    bb = 16

    def body(x_ref, w_ref, m_ref, b_ref, o_ref):
        y = lax.dot_general(
            m_ref[...], x_ref[...].astype(jnp.bfloat16),
            (((1,), (1,)), ((), ())),
            preferred_element_type=jnp.float32)
        acc = y[0:FSIZE, :]
        for p in range(1, NPOS):
            acc = jnp.maximum(acc, y[p * FSIZE:(p + 1) * FSIZE, :])
        acc = jnp.maximum(acc + b_ref[...], 0.0)
        o_ref[...] = jnp.concatenate(
            [w_ref[...], jnp.transpose(acc)],
            axis=1).reshape(bb, QL, OUT)

    return pl.pallas_call(
        body,
        grid=(B // bb,),
        in_specs=[
            pl.BlockSpec((bb * QL, WL * CDIM), lambda i: (i, 0)),
            pl.BlockSpec((bb * QL, EMB), lambda i: (i, 0)),
            pl.BlockSpec((NPOS * FSIZE, WL * CDIM), lambda i: (0, 0)),
            pl.BlockSpec((FSIZE, 1), lambda i: (0, 0)),
        ],
        out_specs=pl.BlockSpec((bb, QL, OUT), lambda i: (i, 0, 0)),
        out_shape=jax.ShapeDtypeStruct((B, QL, OUT), jnp.float32),
    )(x, wemb, Mt, bias)


def _build_band(conv_w):
    wct = jnp.transpose(conv_w[:, :, 0, :], (2, 1, 0))
    eyes = jnp.stack([jnp.eye(WL, NPOS, k=-d, dtype=jnp.float32)
                      for d in range(FWIDTH)])
    m4 = jnp.einsum("dwp,dcf->pfcw", eyes, wct)
    return m4.reshape(NPOS * FSIZE, WL * CDIM)


def kernel(doc_w, doc_c, qry_w, qry_c, k_layer, K, W, char_table, conv_w, conv_b):
    dw = doc_w.astype(jnp.int32).reshape(ND // 128, 128)
    qw = qry_w.astype(jnp.int32).reshape(NQ)
    dcT = jnp.transpose(doc_c.astype(jnp.int32), (0, 2, 1)).reshape(B * WL, DL)
    qcT = jnp.transpose(qry_c.astype(jnp.int32), (0, 2, 1)).reshape(B * WL * QL)
    Wt = W.astype(jnp.float32)
    ctT = char_table.astype(jnp.float32).T

    wdA, cdA, wq, cq = _sc_gather_half(
        Wt, ctT, dw[:ND // 256], dcT[:BH * WL], qw, qcT)
    wdB, cdB = _sc_gather_half(Wt, ctT, dw[ND // 256:], dcT[BH * WL:])

    Mt = _build_band(conv_w.astype(jnp.float32)).astype(jnp.bfloat16)
    bias = conv_b.astype(jnp.float32).reshape(FSIZE, 1)

    outdA = _tc_conv(cdA, wdA, Mt, bias, BH, 0, B)
    outq = _tc_conv_qry(cq, wq, Mt, bias)
    outd = _tc_conv(cdB, wdB, Mt, bias, BH, BH, B, prev=outdA)
    return jnp.transpose(outd, (0, 2, 1)), outq


# speedup vs baseline: 2.3418x; 1.3582x over previous
import functools

import jax
import jax.numpy as jnp
from jax import lax
from jax.experimental import pallas as pl
from jax.experimental.pallas import tpu as pltpu
from jax.experimental.pallas import tpu_sc as plsc

VOCAB = 100000
EMB = 128
NCHAR = 128
CDIM = 16
FSIZE = 64
FWIDTH = 5
B = 64
DL = 512
QL = 32
WL = 16
NPOS = WL - FWIDTH + 1
OUT = EMB + FSIZE

NW = 32
ND = B * DL
NQ = B * QL
BH = B // 2
NDH = BH * DL
CCHUNK = 128
NCH = DL // CCHUNK


def _sc_gather_half(Wt, ctT, dw, dcT, qw=None, qcT=None):
    with_qry = qw is not None
    mesh = plsc.VectorSubcoreMesh(core_axis_name="c", subcore_axis_name="s")
    out_type = [
        jax.ShapeDtypeStruct((NDH, EMB), jnp.float32),
        jax.ShapeDtypeStruct((NDH, WL * CDIM), jnp.float32),
    ]
    if with_qry:
        out_type += [
            jax.ShapeDtypeStruct((NQ, EMB), jnp.float32),
            jax.ShapeDtypeStruct((NQ, WL * CDIM), jnp.float32),
        ]

    def k(*refs):
        if with_qry:
            (w_hbm, ctT_hbm, dw_hbm, dcT_flat, qw_hbm, qcT_flat,
             wd_out, cd_out, wq_out, cq_out,
             idx_v, wrows, ctT_v, ids_v, qids_v, qwi_v, cemb_v,
             sem_g, sem_i, sem_wo, sem_co) = refs
        else:
            (w_hbm, ctT_hbm, dw_hbm, dcT_flat,
             wd_out, cd_out,
             idx_v, wrows, ctT_v, ids_v, qids_v, qwi_v, cemb_v,
             sem_g, sem_i, sem_wo, sem_co) = refs
        wid = lax.axis_index("s") * 2 + lax.axis_index("c")

        pltpu.sync_copy(ctT_hbm, ctT_v)
        pltpu.sync_copy(dw_hbm.at[pl.ds(wid * NCH, NCH)], idx_v)
        lane16 = lax.iota(jnp.int32, 16)
        dvecs = [jnp.full((16,), d, jnp.int32) for d in range(CDIM)]

        def ids_copy(c, buf):
            return pltpu.async_copy(
                dcT_flat.at[pl.ds(wid * WL, WL), pl.ds(c * CCHUNK, CCHUNK)],
                ids_v.at[buf], sem_i)

        def char_tokens(buf):
            @plsc.parallel_loop(0, CCHUNK, 1, unroll=4)
            def tok_body(j):
                ids = plsc.load_gather(
                    ids_v.at[buf], [lane16, jnp.full((16,), j, jnp.int32)])
                for d in range(CDIM):
                    cemb_v[buf, j, pl.ds(d * WL, WL)] = plsc.load_gather(
                        ctT_v, [dvecs[d], ids])

        ids_copy(0, 0).wait()
        ids_pf = ids_copy(1, 1)
        gat = {}
        wrt_w = {}
        wrt_c = {}
        for c in range(NCH):
            h = c % 2
            if c >= 2:
                wrt_w[c - 2].wait()
                wrt_c[c - 2].wait()
            gat[c] = pltpu.async_copy(w_hbm.at[idx_v.at[c]],
                                      wrows.at[h], sem_g)
            if c >= 1:
                ids_pf.wait()
                if c < NCH - 1:
                    ids_pf = ids_copy(c + 1, (c + 1) % 2)
            char_tokens(h)
            off = wid * DL + c * CCHUNK
            wrt_c[c] = pltpu.async_copy(
                cemb_v.at[h], cd_out.at[pl.ds(off, CCHUNK)], sem_co)
            gat[c].wait()
            wrt_w[c] = pltpu.async_copy(
                wrows.at[h], wd_out.at[pl.ds(off, CCHUNK)], sem_wo)
        for c in range(NCH - 2, NCH):
            wrt_w[c].wait()
            wrt_c[c].wait()

        if with_qry:
            pltpu.sync_copy(qw_hbm.at[pl.ds(wid * 64, 64)], qwi_v)
            qw_gat = pltpu.async_copy(w_hbm.at[qwi_v],
                                      wrows.at[0, pl.ds(0, 64)], sem_g)

            def qry_tokens(base):
                @plsc.parallel_loop(0, QL, 1, unroll=4)
                def tok_body(j):
                    ids = plsc.load_gather(qids_v, [lane16 * QL + j])
                    for d in range(CDIM):
                        cemb_v[0, base + j, pl.ds(d * WL, WL)] = (
                            plsc.load_gather(ctT_v, [dvecs[d], ids]))

            for sb in range(2):
                batch = 2 * wid + sb
                pltpu.sync_copy(
                    qcT_flat.at[pl.ds(batch * WL * QL, WL * QL)], qids_v)
                qry_tokens(sb * QL)
            pltpu.sync_copy(cemb_v.at[0, pl.ds(0, 2 * QL)],
                            cq_out.at[pl.ds(2 * wid * QL, 2 * QL)])

            qw_gat.wait()
            pltpu.sync_copy(wrows.at[0, pl.ds(0, 64)],
                            wq_out.at[pl.ds(wid * 64, 64)])

    kk = pl.kernel(
        k,
        mesh=mesh,
        compiler_params=pltpu.CompilerParams(needs_layout_passes=False),
        out_type=out_type,
        scratch_types=[
            pltpu.VMEM((NCH, 128), jnp.int32),
            pltpu.VMEM((2, 128, EMB), jnp.float32),
            pltpu.VMEM((CDIM, NCHAR), jnp.float32),
            pltpu.VMEM((2, WL, CCHUNK), jnp.int32),
            pltpu.VMEM((WL * QL,), jnp.int32),
            pltpu.VMEM((64,), jnp.int32),
            pltpu.VMEM((2, CCHUNK, WL * CDIM), jnp.float32),
            pltpu.SemaphoreType.DMA,
            pltpu.SemaphoreType.DMA,
            pltpu.SemaphoreType.DMA,
            pltpu.SemaphoreType.DMA,
        ],
    )
    if with_qry:
        return kk(Wt, ctT, dw, dcT, qw, qcT)
    return kk(Wt, ctT, dw, dcT)


def _tc_conv(x, wemb, Mt, bias, nb, b_off, nb_total, prev=None):
    def body(x_ref, w_ref, m_ref, b_ref, *rest):
        o_ref = rest[-1]
        y = lax.dot_general(
            m_ref[...], x_ref[...].astype(jnp.bfloat16),
            (((1,), (1,)), ((), ())),
            preferred_element_type=jnp.float32)
        acc = y[0:FSIZE, :]
        for p in range(1, NPOS):
            acc = jnp.maximum(acc, y[p * FSIZE:(p + 1) * FSIZE, :])
        acc = jnp.maximum(acc + b_ref[...], 0.0)
        o_ref[0, 0:EMB, :] = jnp.transpose(w_ref[...])
        o_ref[0, EMB:OUT, :] = acc

    in_specs = [
        pl.BlockSpec((DL, WL * CDIM), lambda i: (i, 0)),
        pl.BlockSpec((DL, EMB), lambda i: (i, 0)),
        pl.BlockSpec((NPOS * FSIZE, WL * CDIM), lambda i: (0, 0)),
        pl.BlockSpec((FSIZE, 1), lambda i: (0, 0)),
    ]
    args = [x, wemb, Mt, bias]
    aliases = {}
    if prev is not None:
        in_specs.append(pl.BlockSpec(memory_space=pl.ANY))
        args.append(prev)
        aliases = {4: 0}
    return pl.pallas_call(
        body,
        grid=(nb,),
        in_specs=in_specs,
        out_specs=pl.BlockSpec((1, OUT, DL), lambda i: (i + b_off, 0, 0)),
        out_shape=jax.ShapeDtypeStruct((nb_total, OUT, DL), jnp.float32),
        input_output_aliases=aliases,
    )(*args)


def _tc_conv_qry(x, wemb, Mt, bias):
    bb = 16

    def body(x_ref, w_ref, m_ref, b_ref, o_ref):
        y = lax.dot_general(
            m_ref[...], x_ref[...].astype(jnp.bfloat16),
            (((1,), (1,)), ((), ())),
            preferred_element_type=jnp.float32)
        acc = y[0:FSIZE, :]
        for p in range(1, NPOS):
            acc = jnp.maximum(acc, y[p * FSIZE:(p + 1) * FSIZE, :])
        acc = jnp.maximum(acc + b_ref[...], 0.0)
        o_ref[...] = jnp.concatenate(
            [w_ref[...], jnp.transpose(acc)],
            axis=1).reshape(bb, QL, OUT)

    return pl.pallas_call(
        body,
        grid=(B // bb,),
        in_specs=[
            pl.BlockSpec((bb * QL, WL * CDIM), lambda i: (i, 0)),
            pl.BlockSpec((bb * QL, EMB), lambda i: (i, 0)),
            pl.BlockSpec((NPOS * FSIZE, WL * CDIM), lambda i: (0, 0)),
            pl.BlockSpec((FSIZE, 1), lambda i: (0, 0)),
        ],
        out_specs=pl.BlockSpec((bb, QL, OUT), lambda i: (i, 0, 0)),
        out_shape=jax.ShapeDtypeStruct((B, QL, OUT), jnp.float32),
    )(x, wemb, Mt, bias)


def _build_band(conv_w):
    wct = jnp.transpose(conv_w[:, :, 0, :], (2, 1, 0))
    eyes = jnp.stack([jnp.eye(WL, NPOS, k=-d, dtype=jnp.float32)
                      for d in range(FWIDTH)])
    m4 = jnp.einsum("dwp,dcf->pfcw", eyes, wct)
    return m4.reshape(NPOS * FSIZE, WL * CDIM)


def kernel(doc_w, doc_c, qry_w, qry_c, k_layer, K, W, char_table, conv_w, conv_b):
    dw = doc_w.astype(jnp.int32).reshape(ND // 128, 128)
    qw = qry_w.astype(jnp.int32).reshape(NQ)
    dcT = jnp.transpose(doc_c.astype(jnp.int32), (0, 2, 1)).reshape(B * WL, DL)
    qcT = jnp.transpose(qry_c.astype(jnp.int32), (0, 2, 1)).reshape(B * WL * QL)
    Wt = W.astype(jnp.float32)
    ctT = char_table.astype(jnp.float32).T

    wdA, cdA, wq, cq = _sc_gather_half(
        Wt, ctT, dw[:ND // 256], dcT[:BH * WL], qw, qcT)
    wdB, cdB = _sc_gather_half(Wt, ctT, dw[ND // 256:], dcT[BH * WL:])

    Mt = _build_band(conv_w.astype(jnp.float32)).astype(jnp.bfloat16)
    bias = conv_b.astype(jnp.float32).reshape(FSIZE, 1)

    outdA = _tc_conv(cdA, wdA, Mt, bias, BH, 0, B)
    outq = _tc_conv_qry(cq, wq, Mt, bias)
    outd = _tc_conv(cdB, wdB, Mt, bias, BH, BH, B, prev=outdA)
    return jnp.transpose(outd, (0, 2, 1)), outq


# speedup vs baseline: 2.7174x; 1.1604x over previous
import functools

import jax
import jax.numpy as jnp
from jax import lax
from jax.experimental import pallas as pl
from jax.experimental.pallas import tpu as pltpu
from jax.experimental.pallas import tpu_sc as plsc

VOCAB = 100000
EMB = 128
NCHAR = 128
CDIM = 16
FSIZE = 64
FWIDTH = 5
B = 64
DL = 512
QL = 32
WL = 16
NPOS = WL - FWIDTH + 1
OUT = EMB + FSIZE

NW = 32
ND = B * DL
NQ = B * QL
BH = B // 2
NDH = BH * DL
CCHUNK = 128
NCH = DL // CCHUNK


def _sc_gather_half(Wt, ctT, dw, dcT, qw=None, qcT=None):
    with_qry = qw is not None
    mesh = plsc.VectorSubcoreMesh(core_axis_name="c", subcore_axis_name="s")
    out_type = [
        jax.ShapeDtypeStruct((NDH, EMB), jnp.float32),
        jax.ShapeDtypeStruct((NDH, WL * CDIM), jnp.float32),
    ]
    if with_qry:
        out_type += [
            jax.ShapeDtypeStruct((NQ, EMB), jnp.float32),
            jax.ShapeDtypeStruct((NQ, WL * CDIM), jnp.float32),
        ]

    def k(*refs):
        if with_qry:
            (w_hbm, ctT_hbm, dw_hbm, dcT_flat, qw_hbm, qcT_flat,
             wd_out, cd_out, wq_out, cq_out,
             idx_v, wrows, ctT_v, ids_v, qids_v, qwi_v, cemb_v,
             sem_g, sem_i, sem_wo, sem_co) = refs
        else:
            (w_hbm, ctT_hbm, dw_hbm, dcT_flat,
             wd_out, cd_out,
             idx_v, wrows, ctT_v, ids_v, qids_v, qwi_v, cemb_v,
             sem_g, sem_i, sem_wo, sem_co) = refs
        wid = lax.axis_index("s") * 2 + lax.axis_index("c")

        pltpu.sync_copy(ctT_hbm, ctT_v)
        pltpu.sync_copy(dw_hbm.at[pl.ds(wid * DL, DL)], idx_v)
        lane16 = lax.iota(jnp.int32, 16)
        dvecs = [jnp.full((16,), d, jnp.int32) for d in range(CDIM)]

        def ids_copy(c, buf):
            return pltpu.async_copy(
                dcT_flat.at[pl.ds(wid * WL, WL), pl.ds(c * CCHUNK, CCHUNK)],
                ids_v.at[buf], sem_i)

        def char_tokens(buf):
            @plsc.parallel_loop(0, CCHUNK, 1, unroll=8)
            def tok_body(j):
                ids = plsc.load_gather(
                    ids_v.at[buf], [lane16, jnp.full((16,), j, jnp.int32)])
                for d in range(CDIM):
                    cemb_v[buf, j, pl.ds(d * WL, WL)] = plsc.load_gather(
                        ctT_v, [dvecs[d], ids])

        ids_copy(0, 0).wait()
        ids_pf = ids_copy(1, 1)
        gat = {}
        wrt_w = {}
        wrt_c = {}
        for c in range(NCH):
            h = c % 2
            if c >= 2:
                wrt_w[c - 2].wait()
                wrt_c[c - 2].wait()
            gat[c] = pltpu.async_copy(
                w_hbm.at[idx_v.at[pl.ds(c * CCHUNK, CCHUNK)]],
                wrows.at[h], sem_g)
            if c >= 1:
                ids_pf.wait()
                if c < NCH - 1:
                    ids_pf = ids_copy(c + 1, (c + 1) % 2)
            char_tokens(h)
            off = wid * DL + c * CCHUNK
            wrt_c[c] = pltpu.async_copy(
                cemb_v.at[h], cd_out.at[pl.ds(off, CCHUNK)], sem_co)
            gat[c].wait()
            wrt_w[c] = pltpu.async_copy(
                wrows.at[h], wd_out.at[pl.ds(off, CCHUNK)], sem_wo)
        for c in range(NCH - 2, NCH):
            wrt_w[c].wait()
            wrt_c[c].wait()

        if with_qry:
            pltpu.sync_copy(qw_hbm.at[pl.ds(wid * 64, 64)], qwi_v)
            qw_gat = pltpu.async_copy(w_hbm.at[qwi_v],
                                      wrows.at[0, pl.ds(0, 64)], sem_g)

            def qry_tokens(base):
                @plsc.parallel_loop(0, QL, 1, unroll=4)
                def tok_body(j):
                    ids = plsc.load_gather(qids_v, [lane16 * QL + j])
                    for d in range(CDIM):
                        cemb_v[0, base + j, pl.ds(d * WL, WL)] = (
                            plsc.load_gather(ctT_v, [dvecs[d], ids]))

            for sb in range(2):
                batch = 2 * wid + sb
                pltpu.sync_copy(
                    qcT_flat.at[pl.ds(batch * WL * QL, WL * QL)], qids_v)
                qry_tokens(sb * QL)
            pltpu.sync_copy(cemb_v.at[0, pl.ds(0, 2 * QL)],
                            cq_out.at[pl.ds(2 * wid * QL, 2 * QL)])

            qw_gat.wait()
            pltpu.sync_copy(wrows.at[0, pl.ds(0, 64)],
                            wq_out.at[pl.ds(wid * 64, 64)])

    kk = pl.kernel(
        k,
        mesh=mesh,
        compiler_params=pltpu.CompilerParams(needs_layout_passes=False),
        out_type=out_type,
        scratch_types=[
            pltpu.VMEM((DL,), jnp.int32),
            pltpu.VMEM((2, 128, EMB), jnp.float32),
            pltpu.VMEM((CDIM, NCHAR), jnp.float32),
            pltpu.VMEM((2, WL, CCHUNK), jnp.int32),
            pltpu.VMEM((WL * QL,), jnp.int32),
            pltpu.VMEM((64,), jnp.int32),
            pltpu.VMEM((2, CCHUNK, WL * CDIM), jnp.float32),
            pltpu.SemaphoreType.DMA,
            pltpu.SemaphoreType.DMA,
            pltpu.SemaphoreType.DMA,
            pltpu.SemaphoreType.DMA,
        ],
    )
    if with_qry:
        return kk(Wt, ctT, dw, dcT, qw, qcT)
    return kk(Wt, ctT, dw, dcT)


def _tc_conv(x, wemb, Mt, bias, nb, b_off, nb_total, prev=None):
    bb = 2

    def body(x_ref, w_ref, m_ref, b_ref, *rest):
        o_ref = rest[-1]
        y = lax.dot_general(
            m_ref[...], x_ref[...].astype(jnp.bfloat16),
            (((1,), (1,)), ((), ())),
            preferred_element_type=jnp.float32)
        acc = y[0:FSIZE, :]
        for p in range(1, NPOS):
            acc = jnp.maximum(acc, y[p * FSIZE:(p + 1) * FSIZE, :])
        acc = jnp.maximum(acc + b_ref[...], 0.0)
        for b in range(bb):
            o_ref[b, 0:EMB, :] = jnp.transpose(
                w_ref[pl.ds(b * DL, DL), :])
            o_ref[b, EMB:OUT, :] = acc[:, b * DL:(b + 1) * DL]

    in_specs = [
        pl.BlockSpec((bb * DL, WL * CDIM), lambda i: (i, 0)),
        pl.BlockSpec((bb * DL, EMB), lambda i: (i, 0)),
        pl.BlockSpec((NPOS * FSIZE, WL * CDIM), lambda i: (0, 0)),
        pl.BlockSpec((FSIZE, 1), lambda i: (0, 0)),
    ]
    args = [x, wemb, Mt, bias]
    aliases = {}
    if prev is not None:
        in_specs.append(pl.BlockSpec(memory_space=pl.ANY))
        args.append(prev)
        aliases = {4: 0}
    return pl.pallas_call(
        body,
        grid=(nb // bb,),
        in_specs=in_specs,
        out_specs=pl.BlockSpec(
            (bb, OUT, DL), lambda i: (i + b_off // bb, 0, 0)),
        out_shape=jax.ShapeDtypeStruct((nb_total, OUT, DL), jnp.float32),
        input_output_aliases=aliases,
    )(*args)


def _tc_conv_qry(x, wemb, Mt, bias):
    bb = 16

    def body(x_ref, w_ref, m_ref, b_ref, o_ref):
        y = lax.dot_general(
            m_ref[...], x_ref[...].astype(jnp.bfloat16),
            (((1,), (1,)), ((), ())),
            preferred_element_type=jnp.float32)
        acc = y[0:FSIZE, :]
        for p in range(1, NPOS):
            acc = jnp.maximum(acc, y[p * FSIZE:(p + 1) * FSIZE, :])
        acc = jnp.maximum(acc + b_ref[...], 0.0)
        o_ref[...] = jnp.concatenate(
            [w_ref[...], jnp.transpose(acc)],
            axis=1).reshape(bb, QL, OUT)

    return pl.pallas_call(
        body,
        grid=(B // bb,),
        in_specs=[
            pl.BlockSpec((bb * QL, WL * CDIM), lambda i: (i, 0)),
            pl.BlockSpec((bb * QL, EMB), lambda i: (i, 0)),
            pl.BlockSpec((NPOS * FSIZE, WL * CDIM), lambda i: (0, 0)),
            pl.BlockSpec((FSIZE, 1), lambda i: (0, 0)),
        ],
        out_specs=pl.BlockSpec((bb, QL, OUT), lambda i: (i, 0, 0)),
        out_shape=jax.ShapeDtypeStruct((B, QL, OUT), jnp.float32),
    )(x, wemb, Mt, bias)


def _build_band(conv_w):
    wct = jnp.transpose(conv_w[:, :, 0, :], (2, 1, 0))
    eyes = jnp.stack([jnp.eye(WL, NPOS, k=-d, dtype=jnp.float32)
                      for d in range(FWIDTH)])
    m4 = jnp.einsum("dwp,dcf->pfcw", eyes, wct)
    return m4.reshape(NPOS * FSIZE, WL * CDIM)


def kernel(doc_w, doc_c, qry_w, qry_c, k_layer, K, W, char_table, conv_w, conv_b):
    dw = doc_w.astype(jnp.int32).reshape(ND)
    qw = qry_w.astype(jnp.int32).reshape(NQ)
    dcT = jnp.transpose(doc_c.astype(jnp.int32), (0, 2, 1)).reshape(B * WL, DL)
    qcT = jnp.transpose(qry_c.astype(jnp.int32), (0, 2, 1)).reshape(B * WL * QL)
    Wt = W.astype(jnp.float32)
    ctT = char_table.astype(jnp.float32).T

    wdA, cdA, wq, cq = _sc_gather_half(
        Wt, ctT, dw[:NDH], dcT[:BH * WL], qw, qcT)
    wdB, cdB = _sc_gather_half(Wt, ctT, dw[NDH:], dcT[BH * WL:])

    Mt = _build_band(conv_w.astype(jnp.float32)).astype(jnp.bfloat16)
    bias = conv_b.astype(jnp.float32).reshape(FSIZE, 1)

    outdA = _tc_conv(cdA, wdA, Mt, bias, BH, 0, B)
    outq = _tc_conv_qry(cq, wq, Mt, bias)
    outd = _tc_conv(cdB, wdB, Mt, bias, BH, BH, B, prev=outdA)
    return jnp.transpose(outd, (0, 2, 1)), outq


# speedup vs baseline: 2.7426x; 1.0092x over previous
import functools

import jax
import jax.numpy as jnp
from jax import lax
from jax.experimental import pallas as pl
from jax.experimental.pallas import tpu as pltpu
from jax.experimental.pallas import tpu_sc as plsc

VOCAB = 100000
EMB = 128
NCHAR = 128
CDIM = 16
FSIZE = 64
FWIDTH = 5
B = 64
DL = 512
QL = 32
WL = 16
NPOS = WL - FWIDTH + 1
OUT = EMB + FSIZE

NW = 32
ND = B * DL
NQ = B * QL
BH = B // 2
NDH = BH * DL
CCHUNK = 128
NCH = DL // CCHUNK


def _sc_gather_half(Wt, ctT, dw, dcT, qw=None, qcT=None):
    with_qry = qw is not None
    mesh = plsc.VectorSubcoreMesh(core_axis_name="c", subcore_axis_name="s")
    out_type = [
        jax.ShapeDtypeStruct((NDH, EMB), jnp.float32),
        jax.ShapeDtypeStruct((NDH, WL * CDIM), jnp.float32),
    ]
    if with_qry:
        out_type += [
            jax.ShapeDtypeStruct((NQ, EMB), jnp.float32),
            jax.ShapeDtypeStruct((NQ, WL * CDIM), jnp.float32),
        ]

    def k(*refs):
        if with_qry:
            (w_hbm, ctT_hbm, dw_hbm, dcT_flat, qw_hbm, qcT_flat,
             wd_out, cd_out, wq_out, cq_out,
             idx_v, wrows, ctT_v, ids_v, qids_v, qwi_v, cemb_v,
             sem_g, sem_i, sem_wo, sem_co) = refs
        else:
            (w_hbm, ctT_hbm, dw_hbm, dcT_flat,
             wd_out, cd_out,
             idx_v, wrows, ctT_v, ids_v, qids_v, qwi_v, cemb_v,
             sem_g, sem_i, sem_wo, sem_co) = refs
        wid = lax.axis_index("s") * 2 + lax.axis_index("c")

        pltpu.sync_copy(ctT_hbm, ctT_v)
        pltpu.sync_copy(dw_hbm.at[pl.ds(wid * DL, DL)], idx_v)
        lane16 = lax.iota(jnp.int32, 16)
        dvecs = [jnp.full((16,), d, jnp.int32) for d in range(CDIM)]

        def ids_copy(c, buf):
            return pltpu.async_copy(
                dcT_flat.at[pl.ds(wid * WL, WL), pl.ds(c * CCHUNK, CCHUNK)],
                ids_v.at[buf], sem_i)

        def char_tokens(buf):
            @plsc.parallel_loop(0, CCHUNK, 1, unroll=4)
            def tok_body(j):
                ids = plsc.load_gather(
                    ids_v.at[buf], [lane16, jnp.full((16,), j, jnp.int32)])
                for d in range(CDIM):
                    cemb_v[buf, j, pl.ds(d * WL, WL)] = plsc.load_gather(
                        ctT_v, [dvecs[d], ids])

        ids_copy(0, 0).wait()
        ids_pf = ids_copy(1, 1)
        gat = {}
        wrt_w = {}
        wrt_c = {}
        for c in range(NCH):
            h = c % 2
            if c >= 2:
                wrt_w[c - 2].wait()
                wrt_c[c - 2].wait()
            gat[c] = pltpu.async_copy(
                w_hbm.at[idx_v.at[pl.ds(c * CCHUNK, CCHUNK)]],
                wrows.at[h], sem_g)
            if c >= 1:
                ids_pf.wait()
                if c < NCH - 1:
                    ids_pf = ids_copy(c + 1, (c + 1) % 2)
            char_tokens(h)
            off = wid * DL + c * CCHUNK
            wrt_c[c] = pltpu.async_copy(
                cemb_v.at[h], cd_out.at[pl.ds(off, CCHUNK)], sem_co)
            gat[c].wait()
            wrt_w[c] = pltpu.async_copy(
                wrows.at[h], wd_out.at[pl.ds(off, CCHUNK)], sem_wo)
        for c in range(NCH - 2, NCH):
            wrt_w[c].wait()
            wrt_c[c].wait()

        if with_qry:
            pltpu.sync_copy(qw_hbm.at[pl.ds(wid * 64, 64)], qwi_v)
            qw_gat = pltpu.async_copy(w_hbm.at[qwi_v],
                                      wrows.at[0, pl.ds(0, 64)], sem_g)

            def qry_tokens(base):
                @plsc.parallel_loop(0, QL, 1, unroll=4)
                def tok_body(j):
                    ids = plsc.load_gather(qids_v, [lane16 * QL + j])
                    for d in range(CDIM):
                        cemb_v[0, base + j, pl.ds(d * WL, WL)] = (
                            plsc.load_gather(ctT_v, [dvecs[d], ids]))

            for sb in range(2):
                batch = 2 * wid + sb
                pltpu.sync_copy(
                    qcT_flat.at[pl.ds(batch * WL * QL, WL * QL)], qids_v)
                qry_tokens(sb * QL)
            pltpu.sync_copy(cemb_v.at[0, pl.ds(0, 2 * QL)],
                            cq_out.at[pl.ds(2 * wid * QL, 2 * QL)])

            qw_gat.wait()
            pltpu.sync_copy(wrows.at[0, pl.ds(0, 64)],
                            wq_out.at[pl.ds(wid * 64, 64)])

    kk = pl.kernel(
        k,
        mesh=mesh,
        compiler_params=pltpu.CompilerParams(needs_layout_passes=False),
        out_type=out_type,
        scratch_types=[
            pltpu.VMEM((DL,), jnp.int32),
            pltpu.VMEM((2, 128, EMB), jnp.float32),
            pltpu.VMEM((CDIM, NCHAR), jnp.float32),
            pltpu.VMEM((2, WL, CCHUNK), jnp.int32),
            pltpu.VMEM((WL * QL,), jnp.int32),
            pltpu.VMEM((64,), jnp.int32),
            pltpu.VMEM((2, CCHUNK, WL * CDIM), jnp.float32),
            pltpu.SemaphoreType.DMA,
            pltpu.SemaphoreType.DMA,
            pltpu.SemaphoreType.DMA,
            pltpu.SemaphoreType.DMA,
        ],
    )
    if with_qry:
        return kk(Wt, ctT, dw, dcT, qw, qcT)
    return kk(Wt, ctT, dw, dcT)


def _tc_conv(x, wemb, Mt, bias, nb, b_off, nb_total, prev=None):
    bb = 2

    def body(x_ref, w_ref, m_ref, b_ref, *rest):
        o_ref = rest[-1]
        y = lax.dot_general(
            m_ref[...], x_ref[...].astype(jnp.bfloat16),
            (((1,), (1,)), ((), ())),
            preferred_element_type=jnp.float32)
        acc = y[0:FSIZE, :]
        for p in range(1, NPOS):
            acc = jnp.maximum(acc, y[p * FSIZE:(p + 1) * FSIZE, :])
        acc = jnp.maximum(acc + b_ref[...], 0.0)
        for b in range(bb):
            o_ref[b, 0:EMB, :] = jnp.transpose(
                w_ref[pl.ds(b * DL, DL), :])
            o_ref[b, EMB:OUT, :] = acc[:, b * DL:(b + 1) * DL]

    in_specs = [
        pl.BlockSpec((bb * DL, WL * CDIM), lambda i: (i, 0)),
        pl.BlockSpec((bb * DL, EMB), lambda i: (i, 0)),
        pl.BlockSpec((NPOS * FSIZE, WL * CDIM), lambda i: (0, 0)),
        pl.BlockSpec((FSIZE, 1), lambda i: (0, 0)),
    ]
    args = [x, wemb, Mt, bias]
    aliases = {}
    if prev is not None:
        in_specs.append(pl.BlockSpec(memory_space=pl.ANY))
        args.append(prev)
        aliases = {4: 0}
    return pl.pallas_call(
        body,
        grid=(nb // bb,),
        in_specs=in_specs,
        out_specs=pl.BlockSpec(
            (bb, OUT, DL), lambda i: (i + b_off // bb, 0, 0)),
        out_shape=jax.ShapeDtypeStruct((nb_total, OUT, DL), jnp.float32),
        input_output_aliases=aliases,
    )(*args)


def _tc_conv_qry(x, wemb, Mt, bias):
    bb = 16

    def body(x_ref, w_ref, m_ref, b_ref, o_ref):
        y = lax.dot_general(
            m_ref[...], x_ref[...].astype(jnp.bfloat16),
            (((1,), (1,)), ((), ())),
            preferred_element_type=jnp.float32)
        acc = y[0:FSIZE, :]
        for p in range(1, NPOS):
            acc = jnp.maximum(acc, y[p * FSIZE:(p + 1) * FSIZE, :])
        acc = jnp.maximum(acc + b_ref[...], 0.0)
        o_ref[...] = jnp.concatenate(
            [w_ref[...], jnp.transpose(acc)],
            axis=1).reshape(bb, QL, OUT)

    return pl.pallas_call(
        body,
        grid=(B // bb,),
        in_specs=[
            pl.BlockSpec((bb * QL, WL * CDIM), lambda i: (i, 0)),
            pl.BlockSpec((bb * QL, EMB), lambda i: (i, 0)),
            pl.BlockSpec((NPOS * FSIZE, WL * CDIM), lambda i: (0, 0)),
            pl.BlockSpec((FSIZE, 1), lambda i: (0, 0)),
        ],
        out_specs=pl.BlockSpec((bb, QL, OUT), lambda i: (i, 0, 0)),
        out_shape=jax.ShapeDtypeStruct((B, QL, OUT), jnp.float32),
    )(x, wemb, Mt, bias)


def _build_band(conv_w):
    wct = jnp.transpose(conv_w[:, :, 0, :], (2, 1, 0))
    eyes = jnp.stack([jnp.eye(WL, NPOS, k=-d, dtype=jnp.float32)
                      for d in range(FWIDTH)])
    m4 = jnp.einsum("dwp,dcf->pfcw", eyes, wct)
    return m4.reshape(NPOS * FSIZE, WL * CDIM)


def kernel(doc_w, doc_c, qry_w, qry_c, k_layer, K, W, char_table, conv_w, conv_b):
    dw = doc_w.astype(jnp.int32).reshape(ND)
    qw = qry_w.astype(jnp.int32).reshape(NQ)
    dcT = jnp.transpose(doc_c.astype(jnp.int32), (0, 2, 1)).reshape(B * WL, DL)
    qcT = jnp.transpose(qry_c.astype(jnp.int32), (0, 2, 1)).reshape(B * WL * QL)
    Wt = W.astype(jnp.float32)
    ctT = char_table.astype(jnp.float32).T

    wdA, cdA, wq, cq = _sc_gather_half(
        Wt, ctT, dw[:NDH], dcT[:BH * WL], qw, qcT)
    wdB, cdB = _sc_gather_half(Wt, ctT, dw[NDH:], dcT[BH * WL:])

    Mt = _build_band(conv_w.astype(jnp.float32)).astype(jnp.bfloat16)
    bias = conv_b.astype(jnp.float32).reshape(FSIZE, 1)

    outdA = _tc_conv(cdA, wdA, Mt, bias, BH, 0, B)
    outq = _tc_conv_qry(cq, wq, Mt, bias)
    outd = _tc_conv(cdB, wdB, Mt, bias, BH, BH, B, prev=outdA)
    return jnp.transpose(outd, (0, 2, 1)), outq
